# Initial kernel scaffold; baseline (speedup 1.0000x reference)
import jax
import jax.numpy as jnp
from jax.experimental import pallas as pl


def kernel(X):
    raise NotImplementedError("write your pallas kernel here")


# speedup vs baseline: 3.9032x; 3.9032x over previous
import jax
import jax.numpy as jnp
from jax.experimental import pallas as pl
from jax.experimental.pallas import tpu as pltpu

_R, _N = 64, 8192


def _entmax_body(x_ref, o_ref, s_ref, st_ref):
    s_ref[:] = x_ref[:] * 0.5
    o_ref[:] = jnp.full((_N, _R), -jnp.inf, dtype=jnp.float32)
    st_ref[0:1, :] = jnp.full((1, _R), jnp.inf, jnp.float32)
    st_ref[1:5, :] = jnp.zeros((4, _R), jnp.float32)
    st_ref[5:6, :] = jnp.ones((1, _R), jnp.float32)

    def cond(state):
        j, go = state
        return jnp.logical_and(j < _N, go > 0)

    def body(state):
        j, _ = state
        t = st_ref[0:1, :]
        c = st_ref[1:2, :]
        cs = st_ref[2:3, :]
        cs2 = st_ref[3:4, :]
        tau_star = st_ref[4:5, :]
        active = st_ref[5:6, :]
        w = s_ref[:]
        cnt_t = jnp.sum((w == t).astype(jnp.float32), axis=0, keepdims=True)
        m_next = jnp.max(jnp.where(w < t, w, -jnp.inf), axis=0, keepdims=True)
        emit_t = c < cnt_t
        m = jnp.where(emit_t, t, m_next)
        c = jnp.where(emit_t, c + 1.0, 1.0)
        rho = (j + 1).astype(jnp.float32)
        cs = cs + m
        cs2 = cs2 + m * m
        mean = cs / rho
        meansq = cs2 / rho
        arg = (1.0 - rho * (meansq - mean * mean)) / rho
        tau = mean - jnp.sqrt(arg)
        keep = jnp.logical_and(active > 0.0, tau <= m)
        keep_f = keep.astype(jnp.float32)
        tau_star = jnp.where(keep, tau, tau_star)
        o_ref[pl.ds(j, 1), :] = m
        st_ref[0:1, :] = m
        st_ref[1:2, :] = c
        st_ref[2:3, :] = cs
        st_ref[3:4, :] = cs2
        st_ref[4:5, :] = tau_star
        st_ref[5:6, :] = keep_f
        n_act = jnp.sum(keep_f)
        return (j + 1, (n_act > 0.0).astype(jnp.int32))

    jax.lax.while_loop(cond, body, (jnp.int32(0), jnp.int32(1)))
    tau_star = st_ref[4:5, :]
    r = jnp.maximum(o_ref[:] - tau_star, 0.0)
    o_ref[:] = r * r


def kernel(X):
    out_t = pl.pallas_call(
        _entmax_body,
        out_shape=jax.ShapeDtypeStruct((_N, _R), jnp.float32),
        scratch_shapes=[
            pltpu.VMEM((_N, _R), jnp.float32),
            pltpu.VMEM((8, _R), jnp.float32),
        ],
    )(X.T)
    return out_t.T


# speedup vs baseline: 6.6732x; 1.7096x over previous
import jax
import jax.numpy as jnp
from jax.experimental import pallas as pl
from jax.experimental.pallas import tpu as pltpu

_R, _N = 64, 8192


def _entmax_body(x_ref, o_ref, s_ref, st_ref):
    s_ref[:] = x_ref[:] * 0.5
    st_ref[:, 0:1] = jnp.full((_R, 1), jnp.inf, jnp.float32)
    st_ref[:, 1:5] = jnp.zeros((_R, 4), jnp.float32)
    st_ref[:, 5:6] = jnp.ones((_R, 1), jnp.float32)

    def cond(state):
        j, go = state
        return jnp.logical_and(j < _N, go > 0)

    def body(state):
        j, _ = state
        t = st_ref[:, 0:1]
        c = st_ref[:, 1:2]
        cs = st_ref[:, 2:3]
        cs2 = st_ref[:, 3:4]
        tau_star = st_ref[:, 4:5]
        active = st_ref[:, 5:6]
        w = s_ref[:]
        cnt_t = jnp.sum((w == t).astype(jnp.float32), axis=1, keepdims=True)
        m_next = jnp.max(jnp.where(w < t, w, -jnp.inf), axis=1, keepdims=True)
        emit_t = c < cnt_t
        m = jnp.where(emit_t, t, m_next)
        c = jnp.where(emit_t, c + 1.0, 1.0)
        rho = (j + 1).astype(jnp.float32)
        cs = cs + m
        cs2 = cs2 + m * m
        mean = cs / rho
        meansq = cs2 / rho
        arg = (1.0 - rho * (meansq - mean * mean)) / rho
        tau = mean - jnp.sqrt(arg)
        keep = jnp.logical_and(active > 0.0, tau <= m)
        keep_f = keep.astype(jnp.float32)
        tau_star = jnp.where(keep, tau, tau_star)
        o_ref[pl.ds(j, 1), :] = m.reshape(1, _R)
        st_ref[:, 0:1] = m
        st_ref[:, 1:2] = c
        st_ref[:, 2:3] = cs
        st_ref[:, 3:4] = cs2
        st_ref[:, 4:5] = tau_star
        st_ref[:, 5:6] = keep_f
        n_act = jnp.sum(keep_f)
        return (j + 1, (n_act > 0.0).astype(jnp.int32))

    j_end, _ = jax.lax.while_loop(cond, body, (jnp.int32(0), jnp.int32(1)))
    tau_star_t = st_ref[:, 4:5].reshape(1, _R)
    row_id = jax.lax.broadcasted_iota(jnp.int32, (_N, _R), 0)
    r = jnp.maximum(o_ref[:] - tau_star_t, 0.0)
    o_ref[:] = jnp.where(row_id < j_end, r * r, 0.0)


def kernel(X):
    out_t = pl.pallas_call(
        _entmax_body,
        out_shape=jax.ShapeDtypeStruct((_N, _R), jnp.float32),
        scratch_shapes=[
            pltpu.VMEM((_R, _N), jnp.float32),
            pltpu.VMEM((_R, 8), jnp.float32),
        ],
    )(X)
    return out_t.T
